# baseline (speedup 1.0000x reference)
import functools

import jax
import jax.numpy as jnp
from jax import lax
from jax.experimental import pallas as pl
from jax.experimental.pallas import tpu as pltpu
from jax.experimental.pallas import tpu_sc as plsc

_S = 4096
_K = 12
_CB = 256
_NBUF = 4
_L = 16
_NCH = _S // _L


def _sc_weights_body(logits_hbm, thres_hbm, w_hbm, o0_v, o1_v, w_v, t_v):
    wid = lax.axis_index("s") * 2 + lax.axis_index("c")

    @pl.when(wid < 16)
    def _task():
        b = wid // 2
        cls = wid % 2
        pltpu.sync_copy(logits_hbm.at[b, 0], o0_v)
        pltpu.sync_copy(logits_hbm.at[b, 1], o1_v)
        pltpu.sync_copy(thres_hbm.at[cls], t_v)
        t = t_v[...]
        pred16 = jnp.broadcast_to(cls == 0, (_L,))
        lane = lax.iota(jnp.int32, _L)

        def body(j, carry):
            acc, top_p, top_i = carry
            off = j * _L
            o0c = o0_v[pl.ds(off, _L)]
            o1c = o1_v[pl.ds(off, _L)]
            m = jnp.maximum(o0c, o1c)
            e0 = jnp.exp(o0c - m)
            e1 = jnp.exp(o1c - m)
            s = e0 + e1
            p = jnp.where(pred16, e1, e0) / s
            wch = jnp.where(p > t, 1.0, 0.0).astype(jnp.float32)
            acc = acc + wch
            w_v[pl.ds(off, _L)] = wch
            sc_, si_ = plsc.sort_key_val(p, lane + off)
            rb = lax.rev(sc_, (0,))
            rbi = lax.rev(si_, (0,))
            ge = top_p >= rb
            merged_p = jnp.where(ge, top_p, rb)
            merged_i = jnp.where(ge, top_i, rbi)
            top_p, top_i = plsc.sort_key_val(merged_p, merged_i)
            return acc, top_p, top_i

        init = (
            jnp.zeros((_L,), jnp.float32),
            jnp.full((_L,), -jnp.inf, jnp.float32),
            jnp.zeros((_L,), jnp.int32),
        )
        acc, top_p, top_i = lax.fori_loop(0, _NCH, body, init)

        cnt = jnp.sum(acc)
        cnt16 = jnp.broadcast_to(cnt, (_L,))
        pos = cnt16 > 0.0
        ones = jnp.full((_L,), jnp.float32(1.0))
        scale16 = jnp.where(pos, ones / jnp.where(pos, cnt16, ones),
                            jnp.zeros((_L,), jnp.float32))

        def norm(j, _):
            off = j * _L
            w_v[pl.ds(off, _L)] = w_v[pl.ds(off, _L)] * scale16
            return 0
        lax.fori_loop(0, _NCH, norm, 0)

        @pl.when(cnt <= 0)
        def _fallback():
            val = jnp.full((_L,), jnp.float32(1.0 / _K))
            plsc.store_scatter(w_v, [top_i], val, mask=lane >= (_L - _K))

        pltpu.sync_copy(w_v, w_hbm.at[b, cls])


def _sc_weights(logits, thres):
    mesh = plsc.VectorSubcoreMesh(core_axis_name="c", subcore_axis_name="s")
    return pl.kernel(
        _sc_weights_body,
        out_type=jax.ShapeDtypeStruct(logits.shape, jnp.float32),
        mesh=mesh,
        compiler_params=pltpu.CompilerParams(needs_layout_passes=False),
        scratch_types=[
            pltpu.VMEM((_S,), jnp.float32),
            pltpu.VMEM((_S,), jnp.float32),
            pltpu.VMEM((_S,), jnp.float32),
            pltpu.VMEM((_L,), jnp.float32),
        ],
    )(logits, thres)


def _make_pool(bs, C):
    nc = C // _CB
    tot = bs * nc

    def pool(w_ref, feat_hbm, out_ref, fbuf, sems):
        def start_copy(c, slot):
            b, jc = divmod(c, nc)
            pltpu.make_async_copy(
                feat_hbm.at[b, pl.ds(jc * _CB, _CB), :],
                fbuf.at[slot],
                sems.at[slot],
            ).start()

        for c in range(min(_NBUF, tot)):
            start_copy(c, c)

        for c in range(tot):
            slot = c % _NBUF
            b, jc = divmod(c, nc)
            pltpu.make_async_copy(
                feat_hbm.at[b, pl.ds(jc * _CB, _CB), :],
                fbuf.at[slot],
                sems.at[slot],
            ).wait()
            res = jax.lax.dot_general(
                w_ref[b], fbuf[slot], (((1,), (1,)), ((), ())),
                preferred_element_type=jnp.float32,
            )
            out_ref[b, :, pl.ds(jc * _CB, _CB)] = res
            if c + _NBUF < tot:
                start_copy(c + _NBUF, slot)

    return pool


@jax.jit
def _run(feature_q, out, tau):
    bs, C = feature_q.shape[0], feature_q.shape[1]
    feat = feature_q.reshape(bs, C, _S)
    logits = out.reshape(bs, 2, _S)
    fg_thres = jax.nn.sigmoid(tau.astype(jnp.float32))
    thres = jnp.stack([
        jnp.full((_L,), fg_thres),
        jnp.full((_L,), 1.0 - fg_thres),
    ])

    w = _sc_weights(logits, thres)

    pool = _make_pool(bs, C)
    protos = pl.pallas_call(
        pool,
        in_specs=[
            pl.BlockSpec((bs, 2, _S), lambda: (0, 0, 0)),
            pl.BlockSpec(memory_space=pltpu.HBM),
        ],
        out_specs=pl.BlockSpec((bs, 2, C), lambda: (0, 0, 0)),
        out_shape=jax.ShapeDtypeStruct((bs, 2, C), jnp.float32),
        scratch_shapes=[
            pltpu.VMEM((_NBUF, _CB, _S), jnp.float32),
            pltpu.SemaphoreType.DMA((_NBUF,)),
        ],
    )(w, feat)

    fg = protos[:, 0, :].reshape(bs, C, 1, 1)
    bg = protos[:, 1, :].reshape(bs, C, 1, 1)
    return fg, bg


def kernel(feature_q, out, tau):
    return _run(feature_q, out, jnp.asarray(tau))
